# baseline (speedup 1.0000x reference)
import jax
import jax.numpy as jnp
from jax import lax
from jax.experimental import pallas as pl
from jax.experimental.pallas import tpu as pltpu
from jax.experimental.pallas import tpu_sc as plsc

N = 10000
D = 128
NT = 10240
NC, NS, L = 2, 16, 16
NW = NC * NS
B = 96
GRP = 36
CHUNKS = 3 * GRP
EPT = B * CHUNKS
E_PAD = EPT * NW

R = 512
G = NT // R


def _logit_head(h, asv, adv, i, as_ref, ad_ref, cv_ref, sm):
    a_s = jnp.sum(h * asv, axis=1)
    a_d = jnp.sum(h * adv, axis=1)
    as_ref[...] = a_s[None, None, :]
    ad_ref[...] = a_d[None, None, :]
    mas = jnp.max(a_s)
    mad = jnp.max(a_d)

    @pl.when(i == 0)
    def _():
        sm[0] = mas
        sm[1] = mad

    @pl.when(i > 0)
    def _():
        sm[0] = jnp.maximum(sm[0], mas)
        sm[1] = jnp.maximum(sm[1], mad)

    @pl.when(i == G - 1)
    def _():
        cs = sm[0] + sm[1]
        cv_ref[...] = jnp.zeros((1, D), jnp.float32) + jnp.maximum(cs, 0.2 * cs)


def _tc_head_body(x_ref, w_ref, asv_ref, adv_ref,
                  h_ref, as_ref, ad_ref, cv_ref, sm):
    h = jnp.dot(x_ref[...], w_ref[...], preferred_element_type=jnp.float32)
    h_ref[...] = h
    _logit_head(h, asv_ref[...], adv_ref[...], pl.program_id(0),
                as_ref, ad_ref, cv_ref, sm)


_LOGIT_OUT_SPECS = [
    pl.BlockSpec((R, D), lambda i: (i, 0)),
    pl.BlockSpec((1, 1, R), lambda i: (i, 0, 0)),
    pl.BlockSpec((1, 1, R), lambda i: (i, 0, 0)),
    pl.BlockSpec((1, D), lambda i: (0, 0)),
]
_LOGIT_OUT_SHAPE = [
    jax.ShapeDtypeStruct((NT, D), jnp.float32),
    jax.ShapeDtypeStruct((G, 1, R), jnp.float32),
    jax.ShapeDtypeStruct((G, 1, R), jnp.float32),
    jax.ShapeDtypeStruct((1, D), jnp.float32),
]


def _tc_head(x, W, a_src, a_dst):
    return pl.pallas_call(
        _tc_head_body,
        grid=(G,),
        in_specs=[
            pl.BlockSpec((R, D), lambda i: (i, 0)),
            pl.BlockSpec((D, D), lambda i: (0, 0)),
            pl.BlockSpec((1, D), lambda i: (0, 0)),
            pl.BlockSpec((1, D), lambda i: (0, 0)),
        ],
        out_specs=_LOGIT_OUT_SPECS,
        out_shape=_LOGIT_OUT_SHAPE,
        scratch_shapes=[pltpu.SMEM((2,), jnp.float32)],
    )(x, W, a_src.reshape(1, D), a_dst.reshape(1, D))


def _tc_mid_body(num_ref, den_ref, b_ref, w_ref, asv_ref, adv_ref,
                 h_ref, as_ref, ad_ref, cv_ref, sm):
    num = num_ref[0] + num_ref[1]
    den = den_ref[0] + den_ref[1]
    y = num / (den + 1e-16)[:, None] + b_ref[...]
    h = jnp.dot(y, w_ref[...], preferred_element_type=jnp.float32)
    h_ref[...] = h
    _logit_head(h, asv_ref[...], adv_ref[...], pl.program_id(0),
                as_ref, ad_ref, cv_ref, sm)


def _tc_mid(num, den, b, W, a_src, a_dst):
    return pl.pallas_call(
        _tc_mid_body,
        grid=(G,),
        in_specs=[
            pl.BlockSpec((2, R, D), lambda i: (0, i, 0)),
            pl.BlockSpec((2, R), lambda i: (0, i)),
            pl.BlockSpec((1, D), lambda i: (0, 0)),
            pl.BlockSpec((D, D), lambda i: (0, 0)),
            pl.BlockSpec((1, D), lambda i: (0, 0)),
            pl.BlockSpec((1, D), lambda i: (0, 0)),
        ],
        out_specs=_LOGIT_OUT_SPECS,
        out_shape=_LOGIT_OUT_SHAPE,
        scratch_shapes=[pltpu.SMEM((2,), jnp.float32)],
    )(num, den, b.reshape(1, D), W, a_src.reshape(1, D), a_dst.reshape(1, D))


def _tc_fin_body(num_ref, den_ref, b_ref, o_ref):
    num = num_ref[0] + num_ref[1]
    den = den_ref[0] + den_ref[1]
    y = num / (den + 1e-16)[:, None] + b_ref[...]
    o_ref[...] = jnp.maximum(y, 0.0)


def _tc_fin(num, den, b):
    return pl.pallas_call(
        _tc_fin_body,
        grid=(G,),
        in_specs=[
            pl.BlockSpec((2, R, D), lambda i: (0, i, 0)),
            pl.BlockSpec((2, R), lambda i: (0, i)),
            pl.BlockSpec((1, D), lambda i: (0, 0)),
        ],
        out_specs=pl.BlockSpec((R, D), lambda i: (i, 0)),
        out_shape=jax.ShapeDtypeStruct((NT, D), jnp.float32),
    )(num, den, b.reshape(1, D))


def _sc_edge_body(h_hbm, eidx_hbm, asrc_hbm, adst_hbm, cub_hbm,
                  num_out, den_out,
                  idxb, didx_b, as_b, ad_b, ex_b, rows_b, cv_v,
                  num_sh, den_sh, isem, gsem, asem, dsem, ssem, esem):
    c = lax.axis_index("c")
    s = lax.axis_index("s")
    wid = c * NS + s

    pltpu.sync_copy(cub_hbm.at[pl.ds(0, L)], cv_v)
    cub = cv_v[pl.ds(0, L)]

    zv = jnp.zeros((L,), jnp.float32)

    @pl.loop(0, B)
    def _zrows(ei):
        for j in range(D // L):
            rows_b[0][ei, pl.ds(j * L, L)] = zv

    for j in range(B // L):
        ex_b[0][pl.ds(j * L, L)] = zv

    for k in range(6):
        pltpu.sync_copy(rows_b[0], num_sh.at[pl.ds(s * 640 + k * B, B)])
        pltpu.sync_copy(ex_b[0], den_sh.at[pl.ds(s * 640 + k * B, B)])
    pltpu.sync_copy(rows_b[0].at[pl.ds(0, 64)],
                    num_sh.at[pl.ds(s * 640 + 576, 64)])
    pltpu.sync_copy(ex_b[0].at[pl.ds(0, 64)],
                    den_sh.at[pl.ds(s * 640 + 576, 64)])
    plsc.subcore_barrier()


    def _fetch_idx(g, islot):
        pltpu.async_copy(eidx_hbm.at[wid, g], idxb[islot], isem[islot])

    def _wait_idx(g, islot):
        pltpu.make_async_copy(eidx_hbm.at[wid, g], idxb[islot],
                              isem[islot]).wait()

    def _gather(islot, k, slot):
        si = idxb[islot].at[k, 0]
        di = idxb[islot].at[k, 1]
        pltpu.async_copy(h_hbm.at[si], rows_b[slot], gsem[slot])
        pltpu.async_copy(asrc_hbm.at[si], as_b[slot], asem[slot])
        pltpu.async_copy(adst_hbm.at[di], ad_b[slot], dsem[slot])

    def _compute(islot, k, slot):
        si = idxb[islot].at[k, 0]
        di = idxb[islot].at[k, 1]
        pltpu.make_async_copy(h_hbm.at[si], rows_b[slot], gsem[slot]).wait()
        pltpu.make_async_copy(asrc_hbm.at[si], as_b[slot], asem[slot]).wait()
        pltpu.make_async_copy(adst_hbm.at[di], ad_b[slot], dsem[slot]).wait()
        for j in range(B // L):
            sl = pl.ds(j * L, L)
            al = as_b[slot][sl] + ad_b[slot][sl]
            al = jnp.maximum(al, 0.2 * al)
            ex_b[slot][sl] = jnp.exp(al - cub)
            didx_b[slot][sl] = idxb[islot][k, 1, sl]

        @pl.loop(0, B // L)
        def _scale(gg):
            exg = ex_b[slot][pl.ds(gg * L, L)]
            for kk in range(L):
                coef = exg[kk]
                ei = gg * L + kk
                for j in range(D // L):
                    sl = pl.ds(j * L, L)
                    rows_b[slot][ei, sl] = rows_b[slot][ei, sl] * coef

    def _scatter_start(slot):
        pltpu.async_copy(rows_b[slot], num_sh.at[didx_b[slot]], ssem[slot],
                         add=True)
        pltpu.async_copy(ex_b[slot], den_sh.at[didx_b[slot]], esem[slot],
                         add=True)

    def _scatter_wait(slot):
        pltpu.make_async_copy(rows_b[slot], num_sh.at[didx_b[slot]],
                              ssem[slot]).wait()
        pltpu.make_async_copy(ex_b[slot], den_sh.at[didx_b[slot]],
                              esem[slot]).wait()

    def _group(g, islot, nislot):
        _compute(islot, 0, 0)

        @pl.when(g > 0)
        def _():
            _scatter_wait(2)

        _gather(islot, 2, 2)
        _scatter_start(0)

        _compute(islot, 1, 1)

        @pl.when(g < GRP - 1)
        def _():
            _wait_idx(g + 1, nislot)
            _scatter_wait(0)
            _gather(nislot, 0, 0)

        _scatter_start(1)

        _compute(islot, 2, 2)

        @pl.when(g < GRP - 2)
        def _():
            _fetch_idx(g + 2, islot)

        @pl.when(g < GRP - 1)
        def _():
            _scatter_wait(1)
            _gather(nislot, 1, 1)

        _scatter_start(2)

    _fetch_idx(0, 0)
    _fetch_idx(1, 1)
    _wait_idx(0, 0)
    _gather(0, 0, 0)
    _gather(0, 1, 1)

    @pl.loop(0, GRP // 2)
    def _ring(gp):
        _group(2 * gp, 0, 1)
        _group(2 * gp + 1, 1, 0)

    for slot in range(3):
        _scatter_wait(slot)

    plsc.subcore_barrier()

    sl = pl.ds(s * 640, 640)
    pltpu.sync_copy(num_sh.at[sl], num_out.at[c, sl])
    pltpu.sync_copy(den_sh.at[sl], den_out.at[c, sl])


_sc_edge = pl.kernel(
    _sc_edge_body,
    out_type=(
        jax.ShapeDtypeStruct((NC, NT, D), jnp.float32),
        jax.ShapeDtypeStruct((NC, NT), jnp.float32),
    ),
    mesh=plsc.VectorSubcoreMesh(core_axis_name="c", subcore_axis_name="s",
                                num_cores=NC, num_subcores=NS),
    compiler_params=pltpu.CompilerParams(needs_layout_passes=False),
    scratch_types=[
        [pltpu.VMEM((3, 2, B), jnp.int32) for _ in range(2)],
        [pltpu.VMEM((B,), jnp.int32) for _ in range(3)],
        [pltpu.VMEM((B,), jnp.float32) for _ in range(3)],
        [pltpu.VMEM((B,), jnp.float32) for _ in range(3)],
        [pltpu.VMEM((B,), jnp.float32) for _ in range(3)],
        [pltpu.VMEM((B, D), jnp.float32) for _ in range(3)],
        pltpu.VMEM((L,), jnp.float32),
        pltpu.VMEM_SHARED((NT, D), jnp.float32),
        pltpu.VMEM_SHARED((NT,), jnp.float32),
        [pltpu.SemaphoreType.DMA for _ in range(2)],
        [pltpu.SemaphoreType.DMA for _ in range(3)],
        [pltpu.SemaphoreType.DMA for _ in range(3)],
        [pltpu.SemaphoreType.DMA for _ in range(3)],
        [pltpu.SemaphoreType.DMA for _ in range(3)],
        [pltpu.SemaphoreType.DMA for _ in range(3)],
    ],
)


def kernel(x, e, W1, a_src1, a_dst1, b1, W2, a_src2, a_dst2, b2):
    src = e[0].astype(jnp.int32)
    dst = e[1].astype(jnp.int32)
    pad = E_PAD - src.shape[0]
    src_p = jnp.concatenate([src, jnp.full((pad,), N, jnp.int32)])
    dst_p = jnp.concatenate([dst, jnp.full((pad,), N, jnp.int32)])
    eidx = jnp.stack([src_p.reshape(NW, GRP, 3, B),
                      dst_p.reshape(NW, GRP, 3, B)], axis=3)
    xp = jnp.pad(x, ((0, NT - N), (0, 0)))

    h1, as1, ad1, cv1 = _tc_head(xp, W1, a_src1, a_dst1)
    num1, den1 = _sc_edge(h1, eidx, as1.reshape(-1), ad1.reshape(-1),
                          cv1.reshape(-1))
    h2, as2, ad2, cv2 = _tc_mid(num1, den1, b1, W2, a_src2, a_dst2)
    num2, den2 = _sc_edge(h2, eidx, as2.reshape(-1), ad2.reshape(-1),
                          cv2.reshape(-1))
    return _tc_fin(num2, den2, b2)[:N]


# speedup vs baseline: 2.9985x; 2.9985x over previous
import jax
import jax.numpy as jnp
from jax import lax
from jax.experimental import pallas as pl
from jax.experimental.pallas import tpu as pltpu
from jax.experimental.pallas import tpu_sc as plsc

N = 10000
D = 128
NT = 10112
NTD = 10240
NC, NS, L = 2, 16, 16
NW = NC * NS
B = 96
GRP = 36
CHUNKS = 3 * GRP
EPT = B * CHUNKS
E_PAD = EPT * NW

R = 128
G = NT // R


def _logit_head(h, asv, adv, i, pk_ref, cv_ref, sm):
    a_s = jnp.sum(h * asv, axis=1)
    a_d = jnp.sum(h * adv, axis=1)
    as_u = lax.bitcast_convert_type(a_s.astype(jnp.bfloat16),
                                    jnp.uint16).astype(jnp.int32)
    ad_u = lax.bitcast_convert_type(a_d.astype(jnp.bfloat16),
                                    jnp.uint16).astype(jnp.int32)
    pk_ref[...] = (jnp.left_shift(ad_u, 16) | as_u)[None, None, :]
    mas = jnp.max(a_s)
    mad = jnp.max(a_d)

    @pl.when(i == 0)
    def _():
        sm[0] = mas
        sm[1] = mad

    @pl.when(i > 0)
    def _():
        sm[0] = jnp.maximum(sm[0], mas)
        sm[1] = jnp.maximum(sm[1], mad)

    @pl.when(i == G - 1)
    def _():
        cs = sm[0] + sm[1]
        cv_ref[...] = jnp.zeros((1, D), jnp.float32) + jnp.maximum(cs, 0.2 * cs)


def _tc_head_body(x_ref, w_ref, asv_ref, adv_ref,
                  h_ref, pk_ref, cv_ref, sm):
    h = jnp.dot(x_ref[...], w_ref[...], preferred_element_type=jnp.float32)
    h_ref[...] = h
    _logit_head(h, asv_ref[...], adv_ref[...], pl.program_id(0),
                pk_ref, cv_ref, sm)


_LOGIT_OUT_SPECS = [
    pl.BlockSpec((R, D), lambda i: (i, 0)),
    pl.BlockSpec((1, 1, R), lambda i: (i, 0, 0)),
    pl.BlockSpec((1, D), lambda i: (0, 0)),
]
_LOGIT_OUT_SHAPE = [
    jax.ShapeDtypeStruct((NT, D), jnp.float32),
    jax.ShapeDtypeStruct((G, 1, R), jnp.int32),
    jax.ShapeDtypeStruct((1, D), jnp.float32),
]


def _tc_head(x, W, a_src, a_dst):
    return pl.pallas_call(
        _tc_head_body,
        grid=(G,),
        in_specs=[
            pl.BlockSpec((R, D), lambda i: (i, 0)),
            pl.BlockSpec((D, D), lambda i: (0, 0)),
            pl.BlockSpec((1, D), lambda i: (0, 0)),
            pl.BlockSpec((1, D), lambda i: (0, 0)),
        ],
        out_specs=_LOGIT_OUT_SPECS,
        out_shape=_LOGIT_OUT_SHAPE,
        scratch_shapes=[pltpu.SMEM((2,), jnp.float32)],
    )(x, W, a_src.reshape(1, D), a_dst.reshape(1, D))


def _tc_mid_body(num_ref, den_ref, b_ref, w_ref, asv_ref, adv_ref,
                 h_ref, pk_ref, cv_ref, sm):
    num = num_ref[0] + num_ref[1]
    den = den_ref[0] + den_ref[1]
    y = num / (den + 1e-16)[:, None] + b_ref[...]
    h = jnp.dot(y, w_ref[...], preferred_element_type=jnp.float32)
    h_ref[...] = h
    _logit_head(h, asv_ref[...], adv_ref[...], pl.program_id(0),
                pk_ref, cv_ref, sm)


def _tc_mid(num, den, b, W, a_src, a_dst):
    return pl.pallas_call(
        _tc_mid_body,
        grid=(G,),
        in_specs=[
            pl.BlockSpec((2, R, D), lambda i: (0, i, 0)),
            pl.BlockSpec((2, R), lambda i: (0, i)),
            pl.BlockSpec((1, D), lambda i: (0, 0)),
            pl.BlockSpec((D, D), lambda i: (0, 0)),
            pl.BlockSpec((1, D), lambda i: (0, 0)),
            pl.BlockSpec((1, D), lambda i: (0, 0)),
        ],
        out_specs=_LOGIT_OUT_SPECS,
        out_shape=_LOGIT_OUT_SHAPE,
        scratch_shapes=[pltpu.SMEM((2,), jnp.float32)],
    )(num, den, b.reshape(1, D), W, a_src.reshape(1, D), a_dst.reshape(1, D))


def _tc_fin_body(num_ref, den_ref, b_ref, o_ref):
    num = num_ref[0] + num_ref[1]
    den = den_ref[0] + den_ref[1]
    y = num / (den + 1e-16)[:, None] + b_ref[...]
    o_ref[...] = jnp.maximum(y, 0.0)


def _tc_fin(num, den, b):
    return pl.pallas_call(
        _tc_fin_body,
        grid=(G,),
        in_specs=[
            pl.BlockSpec((2, R, D), lambda i: (0, i, 0)),
            pl.BlockSpec((2, R), lambda i: (0, i)),
            pl.BlockSpec((1, D), lambda i: (0, 0)),
        ],
        out_specs=pl.BlockSpec((R, D), lambda i: (i, 0)),
        out_shape=jax.ShapeDtypeStruct((NT, D), jnp.float32),
    )(num, den, b.reshape(1, D))


def _sc_edge_body(h_hbm, eidx_hbm, pk_hbm, cub_hbm,
                  num_out, den_out,
                  idxb, didx_b, ex_b, rows_b, pk_v, cv_v,
                  num_sh, den_sh, isem, gsem, ssem, esem):
    c = lax.axis_index("c")
    s = lax.axis_index("s")
    wid = c * NS + s

    pltpu.sync_copy(cub_hbm.at[pl.ds(0, L)], cv_v)
    cub = cv_v[pl.ds(0, L)]
    pltpu.sync_copy(pk_hbm, pk_v)

    zv = jnp.zeros((L,), jnp.float32)

    @pl.loop(0, B)
    def _zrows(ei):
        for j in range(D // L):
            rows_b[0][ei, pl.ds(j * L, L)] = zv

    for j in range(B // L):
        ex_b[0][pl.ds(j * L, L)] = zv

    for k in range(6):
        pltpu.sync_copy(rows_b[0], num_sh.at[pl.ds(s * 632 + k * B, B)])
        pltpu.sync_copy(ex_b[0], den_sh.at[pl.ds(s * 640 + k * B, B)])
    pltpu.sync_copy(rows_b[0].at[pl.ds(0, 56)],
                    num_sh.at[pl.ds(s * 632 + 576, 56)])
    pltpu.sync_copy(ex_b[0].at[pl.ds(0, 64)],
                    den_sh.at[pl.ds(s * 640 + 576, 64)])
    plsc.subcore_barrier()


    def _fetch_idx(g, islot):
        pltpu.async_copy(eidx_hbm.at[wid, g], idxb[islot], isem[islot])

    def _wait_idx(g, islot):
        pltpu.make_async_copy(eidx_hbm.at[wid, g], idxb[islot],
                              isem[islot]).wait()

    def _gather(islot, k, slot):
        si = idxb[islot].at[k, 0]
        pltpu.async_copy(h_hbm.at[si], rows_b[slot], gsem[slot])

    def _compute(islot, k, slot):
        si = idxb[islot].at[k, 0]
        pltpu.make_async_copy(h_hbm.at[si], rows_b[slot], gsem[slot]).wait()
        hi16 = jnp.int32(-65536)
        for j in range(B // L):
            sl = pl.ds(j * L, L)
            siv = idxb[islot][k, 0, sl]
            div = idxb[islot][k, 1, sl]
            ps = plsc.load_gather(pk_v, [siv])
            pd = plsc.load_gather(pk_v, [div])
            a_s = plsc.bitcast(jnp.left_shift(ps, 16), jnp.float32)
            a_d = plsc.bitcast(pd & hi16, jnp.float32)
            al = a_s + a_d
            al = jnp.maximum(al, 0.2 * al)
            ex_b[slot][sl] = jnp.exp(al - cub)
            didx_b[slot][sl] = div

        @pl.loop(0, B // L)
        def _scale(gg):
            exg = ex_b[slot][pl.ds(gg * L, L)]
            for kk in range(L):
                coef = exg[kk]
                ei = gg * L + kk
                for j in range(D // L):
                    sl = pl.ds(j * L, L)
                    rows_b[slot][ei, sl] = rows_b[slot][ei, sl] * coef

    def _scatter_start(slot):
        pltpu.async_copy(rows_b[slot], num_sh.at[didx_b[slot]], ssem[slot],
                         add=True)
        pltpu.async_copy(ex_b[slot], den_sh.at[didx_b[slot]], esem[slot],
                         add=True)

    def _scatter_wait(slot):
        pltpu.make_async_copy(rows_b[slot], num_sh.at[didx_b[slot]],
                              ssem[slot]).wait()
        pltpu.make_async_copy(ex_b[slot], den_sh.at[didx_b[slot]],
                              esem[slot]).wait()

    def _group(g, islot, nislot):
        _compute(islot, 0, 0)

        @pl.when(g > 0)
        def _():
            _scatter_wait(2)

        _gather(islot, 2, 2)
        _scatter_start(0)

        _compute(islot, 1, 1)

        @pl.when(g < GRP - 1)
        def _():
            _wait_idx(g + 1, nislot)
            _scatter_wait(0)
            _gather(nislot, 0, 0)

        _scatter_start(1)

        _compute(islot, 2, 2)

        @pl.when(g < GRP - 2)
        def _():
            _fetch_idx(g + 2, islot)

        @pl.when(g < GRP - 1)
        def _():
            _scatter_wait(1)
            _gather(nislot, 1, 1)

        _scatter_start(2)

    _fetch_idx(0, 0)
    _fetch_idx(1, 1)
    _wait_idx(0, 0)
    _gather(0, 0, 0)
    _gather(0, 1, 1)

    @pl.loop(0, GRP // 2)
    def _ring(gp):
        _group(2 * gp, 0, 1)
        _group(2 * gp + 1, 1, 0)

    for slot in range(3):
        _scatter_wait(slot)

    plsc.subcore_barrier()

    sln = pl.ds(s * 632, 632)
    pltpu.sync_copy(num_sh.at[sln], num_out.at[c, sln])
    sld = pl.ds(s * 640, 640)
    pltpu.sync_copy(den_sh.at[sld], den_out.at[c, sld])


_sc_edge = pl.kernel(
    _sc_edge_body,
    out_type=(
        jax.ShapeDtypeStruct((NC, NT, D), jnp.float32),
        jax.ShapeDtypeStruct((NC, NTD), jnp.float32),
    ),
    mesh=plsc.VectorSubcoreMesh(core_axis_name="c", subcore_axis_name="s",
                                num_cores=NC, num_subcores=NS),
    compiler_params=pltpu.CompilerParams(needs_layout_passes=False),
    scratch_types=[
        [pltpu.VMEM((3, 2, B), jnp.int32) for _ in range(2)],
        [pltpu.VMEM((B,), jnp.int32) for _ in range(3)],
        [pltpu.VMEM((B,), jnp.float32) for _ in range(3)],
        [pltpu.VMEM((B, D), jnp.float32) for _ in range(3)],
        pltpu.VMEM((NT,), jnp.int32),
        pltpu.VMEM((L,), jnp.float32),
        pltpu.VMEM_SHARED((NT, D), jnp.float32),
        pltpu.VMEM_SHARED((NTD,), jnp.float32),
        [pltpu.SemaphoreType.DMA for _ in range(2)],
        [pltpu.SemaphoreType.DMA for _ in range(3)],
        [pltpu.SemaphoreType.DMA for _ in range(3)],
        [pltpu.SemaphoreType.DMA for _ in range(3)],
    ],
)


def kernel(x, e, W1, a_src1, a_dst1, b1, W2, a_src2, a_dst2, b2):
    src = e[0].astype(jnp.int32)
    dst = e[1].astype(jnp.int32)
    pad = E_PAD - src.shape[0]
    fill = N + (jnp.arange(pad, dtype=jnp.int32) % (NT - N))
    src_p = jnp.concatenate([src, fill])
    dst_p = jnp.concatenate([dst, fill])
    eidx = jnp.stack([src_p.reshape(NW, GRP, 3, B),
                      dst_p.reshape(NW, GRP, 3, B)], axis=3)
    xp = jnp.pad(x, ((0, NT - N), (0, 0)))

    h1, pk1, cv1 = _tc_head(xp, W1, a_src1, a_dst1)
    num1, den1 = _sc_edge(h1, eidx, pk1.reshape(-1), cv1.reshape(-1))
    h2, pk2, cv2 = _tc_mid(num1, den1, b1, W2, a_src2, a_dst2)
    num2, den2 = _sc_edge(h2, eidx, pk2.reshape(-1), cv2.reshape(-1))
    return _tc_fin(num2, den2, b2)[:N]


# speedup vs baseline: 3.6551x; 1.2190x over previous
import jax
import jax.numpy as jnp
from jax import lax
from jax.experimental import pallas as pl
from jax.experimental.pallas import tpu as pltpu
from jax.experimental.pallas import tpu_sc as plsc

N = 10000
D = 128
NT = 10112
NTD = 10240
NTC = 10240
NC, NS, L = 2, 16, 16
NW = NC * NS
B = 96
GRP = 36
CHUNKS = 3 * GRP
EPT = B * CHUNKS
E_PAD = EPT * NW

R = 512
G = NTC // R


def _logit_head(h, asv, adv, i, pk_ref, cv_ref, sm):
    a_s = jnp.sum(h * asv, axis=1)
    a_d = jnp.sum(h * adv, axis=1)
    as_u = lax.bitcast_convert_type(a_s.astype(jnp.bfloat16),
                                    jnp.uint16).astype(jnp.int32)
    ad_u = lax.bitcast_convert_type(a_d.astype(jnp.bfloat16),
                                    jnp.uint16).astype(jnp.int32)
    pk_ref[...] = (jnp.left_shift(ad_u, 16) | as_u)[None, None, :]
    mas = jnp.max(a_s)
    mad = jnp.max(a_d)

    @pl.when(i == 0)
    def _():
        sm[0] = mas
        sm[1] = mad

    @pl.when(i > 0)
    def _():
        sm[0] = jnp.maximum(sm[0], mas)
        sm[1] = jnp.maximum(sm[1], mad)

    @pl.when(i == G - 1)
    def _():
        cs = sm[0] + sm[1]
        cv_ref[...] = jnp.zeros((1, D), jnp.float32) + jnp.maximum(cs, 0.2 * cs)


def _tc_head_body(x_ref, w_ref, asv_ref, adv_ref,
                  h_ref, pk_ref, cv_ref, sm):
    h = jnp.dot(x_ref[...], w_ref[...], preferred_element_type=jnp.float32)
    h_ref[...] = h
    _logit_head(h, asv_ref[...], adv_ref[...], pl.program_id(0),
                pk_ref, cv_ref, sm)


_LOGIT_OUT_SPECS = [
    pl.BlockSpec((R, D), lambda i: (i, 0)),
    pl.BlockSpec((1, 1, R), lambda i: (i, 0, 0)),
    pl.BlockSpec((1, D), lambda i: (0, 0)),
]
_LOGIT_OUT_SHAPE = [
    jax.ShapeDtypeStruct((NTC, D), jnp.float32),
    jax.ShapeDtypeStruct((G, 1, R), jnp.int32),
    jax.ShapeDtypeStruct((1, D), jnp.float32),
]


def _tc_head(x, W, a_src, a_dst):
    return pl.pallas_call(
        _tc_head_body,
        grid=(G,),
        in_specs=[
            pl.BlockSpec((R, D), lambda i: (i, 0)),
            pl.BlockSpec((D, D), lambda i: (0, 0)),
            pl.BlockSpec((1, D), lambda i: (0, 0)),
            pl.BlockSpec((1, D), lambda i: (0, 0)),
        ],
        out_specs=_LOGIT_OUT_SPECS,
        out_shape=_LOGIT_OUT_SHAPE,
        scratch_shapes=[pltpu.SMEM((2,), jnp.float32)],
    )(x, W, a_src.reshape(1, D), a_dst.reshape(1, D))


def _tc_mid_body(num_ref, den_ref, b_ref, w_ref, asv_ref, adv_ref,
                 h_ref, pk_ref, cv_ref, sm):
    num = num_ref[0] + num_ref[1]
    den = den_ref[0] + den_ref[1]
    y = num / (den + 1e-16)[:, None] + b_ref[...]
    h = jnp.dot(y, w_ref[...], preferred_element_type=jnp.float32)
    h_ref[...] = h
    _logit_head(h, asv_ref[...], adv_ref[...], pl.program_id(0),
                pk_ref, cv_ref, sm)


def _tc_mid(num, den, b, W, a_src, a_dst):
    return pl.pallas_call(
        _tc_mid_body,
        grid=(G,),
        in_specs=[
            pl.BlockSpec((2, R, D), lambda i: (0, i, 0)),
            pl.BlockSpec((2, R), lambda i: (0, i)),
            pl.BlockSpec((1, D), lambda i: (0, 0)),
            pl.BlockSpec((D, D), lambda i: (0, 0)),
            pl.BlockSpec((1, D), lambda i: (0, 0)),
            pl.BlockSpec((1, D), lambda i: (0, 0)),
        ],
        out_specs=_LOGIT_OUT_SPECS,
        out_shape=_LOGIT_OUT_SHAPE,
        scratch_shapes=[pltpu.SMEM((2,), jnp.float32)],
    )(num, den, b.reshape(1, D), W, a_src.reshape(1, D), a_dst.reshape(1, D))


def _tc_fin_body(num_ref, den_ref, b_ref, o_ref):
    num = num_ref[0] + num_ref[1]
    den = den_ref[0] + den_ref[1]
    y = num / (den + 1e-16)[:, None] + b_ref[...]
    o_ref[...] = jnp.maximum(y, 0.0)


def _tc_fin(num, den, b):
    return pl.pallas_call(
        _tc_fin_body,
        grid=(G,),
        in_specs=[
            pl.BlockSpec((2, R, D), lambda i: (0, i, 0)),
            pl.BlockSpec((2, R), lambda i: (0, i)),
            pl.BlockSpec((1, D), lambda i: (0, 0)),
        ],
        out_specs=pl.BlockSpec((R, D), lambda i: (i, 0)),
        out_shape=jax.ShapeDtypeStruct((NTC, D), jnp.float32),
    )(num, den, b.reshape(1, D))


def _sc_edge_body(h_hbm, eidx_hbm, pk_hbm, cub_hbm,
                  num_out, den_out,
                  idxb, didx_b, ex_b, rows_b, pk_v, cv_v,
                  num_sh, den_sh, isem, gsem, ssem, esem):
    c = lax.axis_index("c")
    s = lax.axis_index("s")
    wid = c * NS + s

    pltpu.sync_copy(cub_hbm.at[pl.ds(0, L)], cv_v)
    cub = cv_v[pl.ds(0, L)]
    pltpu.sync_copy(pk_hbm, pk_v)

    zv = jnp.zeros((L,), jnp.float32)

    @pl.loop(0, B)
    def _zrows(ei):
        for j in range(D // L):
            rows_b[0][ei, pl.ds(j * L, L)] = zv

    for j in range(B // L):
        ex_b[0][pl.ds(j * L, L)] = zv

    for k in range(6):
        pltpu.sync_copy(rows_b[0], num_sh.at[pl.ds(s * 632 + k * B, B)])
        pltpu.sync_copy(ex_b[0], den_sh.at[pl.ds(s * 640 + k * B, B)])
    pltpu.sync_copy(rows_b[0].at[pl.ds(0, 56)],
                    num_sh.at[pl.ds(s * 632 + 576, 56)])
    pltpu.sync_copy(ex_b[0].at[pl.ds(0, 64)],
                    den_sh.at[pl.ds(s * 640 + 576, 64)])
    plsc.subcore_barrier()


    def _fetch_idx(g, islot):
        pltpu.async_copy(eidx_hbm.at[wid, g], idxb[islot], isem[islot])

    def _wait_idx(g, islot):
        pltpu.make_async_copy(eidx_hbm.at[wid, g], idxb[islot],
                              isem[islot]).wait()

    def _gather(islot, k, slot):
        si = idxb[islot].at[k, 0]
        pltpu.async_copy(h_hbm.at[si], rows_b[slot], gsem[slot])

    def _compute(islot, k, slot):
        si = idxb[islot].at[k, 0]
        pltpu.make_async_copy(h_hbm.at[si], rows_b[slot], gsem[slot]).wait()
        hi16 = jnp.int32(-65536)
        for j in range(B // L):
            sl = pl.ds(j * L, L)
            siv = idxb[islot][k, 0, sl]
            div = idxb[islot][k, 1, sl]
            ps = plsc.load_gather(pk_v, [siv])
            pd = plsc.load_gather(pk_v, [div])
            a_s = plsc.bitcast(jnp.left_shift(ps, 16), jnp.float32)
            a_d = plsc.bitcast(pd & hi16, jnp.float32)
            al = a_s + a_d
            al = jnp.maximum(al, 0.2 * al)
            ex_b[slot][sl] = jnp.exp(al - cub)
            didx_b[slot][sl] = div

        @pl.loop(0, B // L)
        def _scale(gg):
            exg = ex_b[slot][pl.ds(gg * L, L)]
            for kk in range(L):
                coef = exg[kk]
                ei = gg * L + kk
                for j in range(D // L):
                    sl = pl.ds(j * L, L)
                    rows_b[slot][ei, sl] = rows_b[slot][ei, sl] * coef

    def _scatter_start(slot):
        pltpu.async_copy(rows_b[slot], num_sh.at[didx_b[slot]], ssem[slot],
                         add=True)
        pltpu.async_copy(ex_b[slot], den_sh.at[didx_b[slot]], esem[slot],
                         add=True)

    def _scatter_wait(slot):
        pltpu.make_async_copy(rows_b[slot], num_sh.at[didx_b[slot]],
                              ssem[slot]).wait()
        pltpu.make_async_copy(ex_b[slot], den_sh.at[didx_b[slot]],
                              esem[slot]).wait()

    def _group(g, islot, nislot):
        _compute(islot, 0, 0)

        @pl.when(g > 0)
        def _():
            _scatter_wait(2)

        _gather(islot, 2, 2)
        _scatter_start(0)

        _compute(islot, 1, 1)

        @pl.when(g < GRP - 1)
        def _():
            _wait_idx(g + 1, nislot)
            _scatter_wait(0)
            _gather(nislot, 0, 0)

        _scatter_start(1)

        _compute(islot, 2, 2)

        @pl.when(g < GRP - 2)
        def _():
            _fetch_idx(g + 2, islot)

        @pl.when(g < GRP - 1)
        def _():
            _scatter_wait(1)
            _gather(nislot, 1, 1)

        _scatter_start(2)

    _fetch_idx(0, 0)
    _fetch_idx(1, 1)
    _wait_idx(0, 0)
    _gather(0, 0, 0)
    _gather(0, 1, 1)

    @pl.loop(0, GRP // 2)
    def _ring(gp):
        _group(2 * gp, 0, 1)
        _group(2 * gp + 1, 1, 0)

    for slot in range(3):
        _scatter_wait(slot)

    plsc.subcore_barrier()

    sln = pl.ds(s * 632, 632)
    pltpu.sync_copy(num_sh.at[sln], num_out.at[c, sln])
    sld = pl.ds(s * 640, 640)
    pltpu.sync_copy(den_sh.at[sld], den_out.at[c, sld])


_sc_edge = pl.kernel(
    _sc_edge_body,
    out_type=(
        jax.ShapeDtypeStruct((NC, NT, D), jnp.float32),
        jax.ShapeDtypeStruct((NC, NTD), jnp.float32),
    ),
    mesh=plsc.VectorSubcoreMesh(core_axis_name="c", subcore_axis_name="s",
                                num_cores=NC, num_subcores=NS),
    compiler_params=pltpu.CompilerParams(needs_layout_passes=False),
    scratch_types=[
        [pltpu.VMEM((3, 2, B), jnp.int32) for _ in range(2)],
        [pltpu.VMEM((B,), jnp.int32) for _ in range(3)],
        [pltpu.VMEM((B,), jnp.float32) for _ in range(3)],
        [pltpu.VMEM((B, D), jnp.float32) for _ in range(3)],
        pltpu.VMEM((NT,), jnp.int32),
        pltpu.VMEM((L,), jnp.float32),
        pltpu.VMEM_SHARED((NT, D), jnp.float32),
        pltpu.VMEM_SHARED((NTD,), jnp.float32),
        [pltpu.SemaphoreType.DMA for _ in range(2)],
        [pltpu.SemaphoreType.DMA for _ in range(3)],
        [pltpu.SemaphoreType.DMA for _ in range(3)],
        [pltpu.SemaphoreType.DMA for _ in range(3)],
    ],
)


def kernel(x, e, W1, a_src1, a_dst1, b1, W2, a_src2, a_dst2, b2):
    src = e[0].astype(jnp.int32)
    dst = e[1].astype(jnp.int32)
    pad = E_PAD - src.shape[0]
    fill = N + (jnp.arange(pad, dtype=jnp.int32) % (NT - N))
    src_p = jnp.concatenate([src, fill])
    dst_p = jnp.concatenate([dst, fill])
    eidx = jnp.stack([src_p.reshape(NW, GRP, 3, B),
                      dst_p.reshape(NW, GRP, 3, B)], axis=3)
    xp = jnp.pad(x, ((0, NTC - N), (0, 0)))

    def padnum(num):
        return jnp.pad(num, ((0, 0), (0, NTC - NT), (0, 0)))

    h1, pk1, cv1 = _tc_head(xp, W1, a_src1, a_dst1)
    num1, den1 = _sc_edge(h1, eidx, pk1.reshape(-1)[:NT], cv1.reshape(-1))
    h2, pk2, cv2 = _tc_mid(padnum(num1), den1, b1, W2, a_src2, a_dst2)
    num2, den2 = _sc_edge(h2, eidx, pk2.reshape(-1)[:NT], cv2.reshape(-1))
    return _tc_fin(padnum(num2), den2, b2)[:N]
